# baseline (speedup 1.0000x reference)
import functools

import jax
import jax.numpy as jnp
from jax import lax
from jax.experimental import pallas as pl
from jax.experimental.pallas import tpu as pltpu
from jax.experimental.pallas import tpu_sc as plsc

EMBED = 64
BATCH = 16384
NUM_ROWS = 1000000
NUM_CORES = 2
NUM_SUBCORES = 16
NUM_WORKERS = NUM_CORES * NUM_SUBCORES
ROWS_PER_WORKER = BATCH // NUM_WORKERS

BLK = 128
FULL_BLKS = NUM_ROWS // BLK
TAIL_LO = FULL_BLKS * BLK
TAIL_N = NUM_ROWS - TAIL_LO
BLKS_PER_W = 245
NBUF = 8
OCTETS = (BLKS_PER_W + NBUF - 1) // NBUF
CAP = 1024
PBITS = 14

_mesh = plsc.VectorSubcoreMesh(core_axis_name="c", subcore_axis_name="s")
_params = pltpu.CompilerParams(needs_layout_passes=False)


@functools.partial(
    pl.kernel,
    mesh=_mesh,
    compiler_params=_params,
    out_type=jax.ShapeDtypeStruct((BATCH, EMBED), jnp.float32),
    scratch_types=[
        pltpu.VMEM((CAP,), jnp.int32),
        pltpu.VMEM((256,), jnp.int32),
        *[pltpu.VMEM((EMBED, BLK), jnp.float32) for _ in range(NBUF)],
        pltpu.VMEM((TAIL_N, EMBED), jnp.float32),
        pltpu.VMEM((8, EMBED), jnp.float32),
        *[pltpu.SemaphoreType.DMA for _ in range(NBUF)],
        pltpu.SemaphoreType.DMA,
    ],
)
def _sc_stream_gather(keys_hbm, bstart_hbm, table_hbm, tail_hbm,
                      stag_hbm, kbuf, bsbuf, *rest):
    bufs = rest[:NBUF]
    tailb = rest[NBUF]
    ring = rest[NBUF + 1]
    sems = rest[NBUF + 2:NBUF + 2 + NBUF]
    semw = rest[NBUF + 2 + NBUF]

    wid = lax.axis_index("s") * NUM_CORES + lax.axis_index("c")
    blk0 = wid * BLKS_PER_W
    jend = jnp.minimum(blk0 + BLKS_PER_W, FULL_BLKS)

    pltpu.sync_copy(bstart_hbm.at[wid], bsbuf)

    lanes = lax.iota(jnp.int32, 16)
    c16 = [lanes + 16 * t for t in range(EMBED // 16)]

    def fetch(buf, lp, size):
        lp = jnp.minimum(lp, size - 1)
        wstart = pl.multiple_of((lp >> 4) << 4, 16)
        win = buf[pl.ds(wstart, 16)]
        sel = win.at[jnp.full((16,), lp & 15, jnp.int32)].get(
            mode="promise_in_bounds")
        return sel[0]

    s0 = fetch(bsbuf, 0, 256)
    a0 = pl.multiple_of((s0 >> 4) << 4, 16)
    pltpu.sync_copy(keys_hbm.at[pl.ds(a0, CAP)], kbuf)
    pltpu.sync_copy(tail_hbm, tailb)

    def blk_bounds(jj):
        s = fetch(bsbuf, jj, 256) - a0
        e = fetch(bsbuf, jj + 1, 256) - a0
        return s, jnp.minimum(e, CAP)

    def issue(jj, buf, sem):
        j = blk0 + jj
        s, e = blk_bounds(jj)

        @pl.when(jnp.logical_and(j < jend, e > s))
        def _():
            src = table_hbm.at[:, pl.ds(pl.multiple_of(j * BLK, BLK), BLK)]
            pltpu.async_copy(src, buf, sem)

    def ring_drain():
        pltpu.make_async_copy(ring.at[pl.ds(0, 1), :],
                              stag_hbm.at[pl.ds(0, 1), :], semw).wait()

    def consume(jj, buf, sem, em):
        j = blk0 + jj
        s, e = blk_bounds(jj)

        def go(em):
            pltpu.make_async_copy(
                table_hbm.at[:, pl.ds(0, BLK)], buf, sem).wait()

            def body(lp, em):
                key = fetch(kbuf, lp, CAP)
                b = key & (BATCH - 1)
                k = (key >> PBITS) - j * BLK
                slot = em & 7

                @pl.when(em >= 8)
                def _():
                    ring_drain()

                kvec = jnp.full((16,), k, jnp.int32)
                for t in range(EMBED // 16):
                    vec = plsc.load_gather(buf, [c16[t], kvec])
                    ring[slot, pl.ds(16 * t, 16)] = vec
                pltpu.async_copy(ring.at[pl.ds(slot, 1), :],
                                 stag_hbm.at[pl.ds(b, 1), :], semw)
                return em + 1

            return lax.fori_loop(s, e, body, em)

        return lax.cond(jnp.logical_and(j < jend, e > s), go,
                        lambda x: x, em)

    for i in range(NBUF):
        issue(i, bufs[i], sems[i])

    def octet_body(t, em):
        for i in range(NBUF):
            jj = t * NBUF + i
            em = consume(jj, bufs[i], sems[i], em)
            issue(jj + NBUF, bufs[i], sems[i])
        return em

    em = lax.fori_loop(0, OCTETS, octet_body, jnp.int32(0))

    tail_s, tail_e = blk_bounds(jend - blk0)

    def tail_go(em):
        def tail_body(lp, em):
            key = fetch(kbuf, lp, CAP)
            b = key & (BATCH - 1)
            k = (key >> PBITS) - TAIL_LO

            @pl.when(em >= 8)
            def _():
                ring_drain()

            pltpu.async_copy(tailb.at[pl.ds(k, 1), :],
                             stag_hbm.at[pl.ds(b, 1), :], semw)
            return em + 1

        return lax.fori_loop(tail_s, tail_e, tail_body, em)

    em = lax.cond(
        jnp.logical_and(wid == NUM_WORKERS - 1, tail_e > tail_s),
        tail_go, lambda x: x, em)

    def drain_body(i, em):
        ring_drain()
        return em

    lax.fori_loop(0, jnp.minimum(em, 8), drain_body, em)


@functools.partial(
    pl.kernel,
    mesh=_mesh,
    compiler_params=_params,
    out_type=jax.ShapeDtypeStruct((BATCH,), jnp.float32),
    scratch_types=[
        pltpu.VMEM((ROWS_PER_WORKER // 2, EMBED), jnp.float32),
        pltpu.VMEM((ROWS_PER_WORKER // 2, EMBED), jnp.float32),
        pltpu.VMEM((ROWS_PER_WORKER,), jnp.float32),
        pltpu.SemaphoreType.DMA,
    ],
)
def _sc_dot(urows_hbm, vrows_hbm, out_hbm, ubuf, vbuf, obuf, sem):
    wid = lax.axis_index("s") * NUM_CORES + lax.axis_index("c")
    base = pl.multiple_of(wid * ROWS_PER_WORKER, ROWS_PER_WORKER)
    half = ROWS_PER_WORKER // 2
    lanes = lax.iota(jnp.int32, 16)

    for p in range(2):
        poff = p * half
        rsl = pl.ds(base + poff, half)
        cu = pltpu.async_copy(urows_hbm.at[rsl, :], ubuf, sem)
        cv = pltpu.async_copy(vrows_hbm.at[rsl, :], vbuf, sem)
        cu.wait()
        cv.wait()

        def group_body(g, carry):
            gbase = pl.multiple_of(g * 16, 16)
            rows16 = gbase + lanes
            acc = jnp.zeros((16,), jnp.float32)
            for d in range(EMBED):
                col = jnp.full((16,), d, jnp.int32)
                acc = acc + (plsc.load_gather(ubuf, [rows16, col])
                             * plsc.load_gather(vbuf, [rows16, col]))
            obuf[pl.ds(poff + gbase, 16)] = acc
            return carry

        lax.fori_loop(0, half // 16, group_body, 0)

    pltpu.sync_copy(obuf, out_hbm.at[pl.ds(base, ROWS_PER_WORKER)])


def _prep(idx):
    packed = (idx << PBITS) | jnp.arange(BATCH, dtype=jnp.int32)
    s = jnp.sort(packed)
    rows_sorted = s >> PBITS
    q = jnp.arange(NUM_WORKERS * 256, dtype=jnp.int32)
    w, o = q // 256, q % 256
    blk_of_q = jnp.minimum(w * BLKS_PER_W + o, FULL_BLKS + 1)
    starts = jnp.searchsorted(rows_sorted, blk_of_q * BLK).astype(jnp.int32)
    bstarts = starts.reshape(NUM_WORKERS, 256)
    sp = jnp.concatenate([s, jnp.full((CAP,), 0x7FFFFFFF, jnp.int32)])
    return sp, bstarts


def kernel(nodes_u, nodes_v, user_emb, item_emb):
    nu = nodes_u.astype(jnp.int32)
    nv = nodes_v.astype(jnp.int32)
    su, bu = _prep(nu)
    sv, bv = _prep(nv)
    tail_u = user_emb[TAIL_LO:]
    tail_v = item_emb[TAIL_LO:]
    stag_u = _sc_stream_gather(su, bu, user_emb.T, tail_u)
    stag_v = _sc_stream_gather(sv, bv, item_emb.T, tail_v)
    out = _sc_dot(stag_u, stag_v)
    return out.reshape(BATCH, 1, 1)


# speedup vs baseline: 5.2486x; 5.2486x over previous
import functools

import jax
import jax.numpy as jnp
from jax import lax
from jax.experimental import pallas as pl
from jax.experimental.pallas import tpu as pltpu
from jax.experimental.pallas import tpu_sc as plsc

EMBED = 64
BATCH = 16384
NUM_ROWS = 1000000
NUM_CORES = 2
NUM_SUBCORES = 16
NUM_WORKERS = NUM_CORES * NUM_SUBCORES
ROWS_PER_WORKER = BATCH // NUM_WORKERS

BLK = 128
FULL_BLKS = NUM_ROWS // BLK
TAIL_LO = FULL_BLKS * BLK
TAIL_N = NUM_ROWS - TAIL_LO
BLKS_PER_W = 245
NBUF = 8
ROUNDS = (BLKS_PER_W + NBUF - 1) // NBUF
ROWS_PER_W_RANGE = BLKS_PER_W * BLK
CAP = 1024
SENT = 0x3FFFFFFF

_mesh = plsc.VectorSubcoreMesh(core_axis_name="c", subcore_axis_name="s")
_params = pltpu.CompilerParams(needs_layout_passes=False)


@functools.partial(
    pl.kernel,
    mesh=_mesh,
    compiler_params=_params,
    out_type=jax.ShapeDtypeStruct((BATCH, EMBED), jnp.float32),
    scratch_types=[
        pltpu.VMEM((16,), jnp.int32),
        pltpu.VMEM((CAP,), jnp.int32),
        pltpu.VMEM((CAP,), jnp.int32),
        *[pltpu.VMEM((EMBED, BLK), jnp.float32) for _ in range(NBUF)],
        pltpu.VMEM((TAIL_N, EMBED), jnp.float32),
        pltpu.VMEM((8, EMBED), jnp.float32),
        *[pltpu.SemaphoreType.DMA for _ in range(NBUF)],
        pltpu.SemaphoreType.DMA,
    ],
)
def _sc_stream_gather(srt_hbm, pos_hbm, bounds_hbm, table_hbm, tail_hbm,
                      stag_hbm, bvec, rbuf, pbuf, *rest):
    bufs = rest[:NBUF]
    tailb = rest[NBUF]
    ring = rest[NBUF + 1]
    sems = rest[NBUF + 2:NBUF + 2 + NBUF]
    semw = rest[NBUF + 2 + NBUF]

    wid = lax.axis_index("s") * NUM_CORES + lax.axis_index("c")
    blk0 = wid * BLKS_PER_W
    jend = jnp.minimum(blk0 + BLKS_PER_W, FULL_BLKS)

    pltpu.sync_copy(bounds_hbm.at[wid], bvec)
    pltpu.sync_copy(tail_hbm, tailb)
    bv = bvec[pl.ds(0, 16)]
    s0 = bv[0]
    a0 = pl.multiple_of((s0 >> 4) << 4, 16)
    pltpu.sync_copy(srt_hbm.at[pl.ds(a0, CAP)], rbuf)
    pltpu.sync_copy(pos_hbm.at[pl.ds(a0, CAP)], pbuf)

    lanes = lax.iota(jnp.int32, 16)
    c16 = [lanes + 16 * t for t in range(EMBED // 16)]

    def fetch(buf, lp):
        lp = jnp.minimum(lp, CAP - 1)
        wstart = pl.multiple_of((lp >> 4) << 4, 16)
        win = buf[pl.ds(wstart, 16)]
        sel = win.at[jnp.full((16,), lp & 15, jnp.int32)].get(
            mode="promise_in_bounds")
        return sel[0]

    def issue(jj, buf, sem):
        j = blk0 + jj

        @pl.when(j < jend)
        def _():
            src = table_hbm.at[:, pl.ds(pl.multiple_of(j * BLK, BLK), BLK)]
            pltpu.async_copy(src, buf, sem)

    def ring_drain():
        pltpu.make_async_copy(ring.at[pl.ds(0, 1), :],
                              stag_hbm.at[pl.ds(0, 1), :], semw).wait()

    def consume(jj, buf, sem, state):
        j = blk0 + jj

        def go(st):
            pltpu.make_async_copy(
                table_hbm.at[:, pl.ds(0, BLK)], buf, sem).wait()

            def cond_fn(st):
                lp, em = st
                return jnp.logical_and(lp < CAP,
                                       fetch(rbuf, lp) < (j + 1) * BLK)

            def body_fn(st):
                lp, em = st
                r = fetch(rbuf, lp)
                b = fetch(pbuf, lp)
                k = r - j * BLK

                def emit(em):
                    slot = em & 7

                    @pl.when(em >= 8)
                    def _():
                        ring_drain()

                    kvec = jnp.full((16,), k, jnp.int32)
                    for t in range(EMBED // 16):
                        vec = plsc.load_gather(buf, [c16[t], kvec])
                        ring[slot, pl.ds(16 * t, 16)] = vec
                    pltpu.async_copy(ring.at[pl.ds(slot, 1), :],
                                     stag_hbm.at[pl.ds(b, 1), :], semw)
                    return em + 1

                em = lax.cond(k >= 0, emit, lambda e: e, em)
                return (lp + 1, em)

            return lax.while_loop(cond_fn, body_fn, st)

        return lax.cond(j < jend, go, lambda s: s, state)

    for i in range(NBUF):
        issue(i, bufs[i], sems[i])

    def round_body(t, state):
        for i in range(NBUF):
            jj = t * NBUF + i
            state = consume(jj, bufs[i], sems[i], state)
            issue(jj + NBUF, bufs[i], sems[i])
        return state

    lp0 = s0 - a0
    state = lax.fori_loop(0, ROUNDS, round_body, (lp0, jnp.int32(0)))

    def tail_cond(st):
        lp, em = st
        r = fetch(rbuf, lp)
        return jnp.logical_and(
            lp < CAP, jnp.logical_and(r >= TAIL_LO, r < SENT))

    def tail_body(st):
        lp, em = st
        r = fetch(rbuf, lp)
        b = fetch(pbuf, lp)
        k = r - TAIL_LO

        @pl.when(em >= 8)
        def _():
            ring_drain()

        pltpu.async_copy(tailb.at[pl.ds(k, 1), :],
                         stag_hbm.at[pl.ds(b, 1), :], semw)
        return (lp + 1, em + 1)

    state = lax.while_loop(tail_cond, tail_body, state)

    def drain_body(i, em):
        ring_drain()
        return em

    lax.fori_loop(0, jnp.minimum(state[1], 8), drain_body, state[1])


@functools.partial(
    pl.kernel,
    mesh=_mesh,
    compiler_params=_params,
    out_type=jax.ShapeDtypeStruct((BATCH,), jnp.float32),
    scratch_types=[
        pltpu.VMEM((ROWS_PER_WORKER // 2, EMBED), jnp.float32),
        pltpu.VMEM((ROWS_PER_WORKER // 2, EMBED), jnp.float32),
        pltpu.VMEM((ROWS_PER_WORKER,), jnp.float32),
        pltpu.SemaphoreType.DMA,
    ],
)
def _sc_dot(urows_hbm, vrows_hbm, out_hbm, ubuf, vbuf, obuf, sem):
    wid = lax.axis_index("s") * NUM_CORES + lax.axis_index("c")
    base = pl.multiple_of(wid * ROWS_PER_WORKER, ROWS_PER_WORKER)
    half = ROWS_PER_WORKER // 2
    lanes = lax.iota(jnp.int32, 16)

    for p in range(2):
        poff = p * half
        rsl = pl.ds(base + poff, half)
        cu = pltpu.async_copy(urows_hbm.at[rsl, :], ubuf, sem)
        cv = pltpu.async_copy(vrows_hbm.at[rsl, :], vbuf, sem)
        cu.wait()
        cv.wait()

        def group_body(g, carry):
            gbase = pl.multiple_of(g * 16, 16)
            rows16 = gbase + lanes
            acc = jnp.zeros((16,), jnp.float32)
            for d in range(EMBED):
                col = jnp.full((16,), d, jnp.int32)
                acc = acc + (plsc.load_gather(ubuf, [rows16, col])
                             * plsc.load_gather(vbuf, [rows16, col]))
            obuf[pl.ds(poff + gbase, 16)] = acc
            return carry

        lax.fori_loop(0, half // 16, group_body, 0)

    pltpu.sync_copy(obuf, out_hbm.at[pl.ds(base, ROWS_PER_WORKER)])


def _prep(idx):
    s, p = lax.sort_key_val(idx, jnp.arange(BATCH, dtype=jnp.int32))
    sp = jnp.concatenate([s, jnp.full((CAP,), SENT, jnp.int32)])
    pp = jnp.concatenate([p, jnp.zeros((CAP,), jnp.int32)])
    r0s = jnp.arange(NUM_WORKERS, dtype=jnp.int32) * ROWS_PER_W_RANGE
    s0s = jnp.searchsorted(s, r0s).astype(jnp.int32)
    bounds = jnp.zeros((NUM_WORKERS, 16), jnp.int32).at[:, 0].set(s0s)
    return sp, pp, bounds


def kernel(nodes_u, nodes_v, user_emb, item_emb):
    nu = nodes_u.astype(jnp.int32)
    nv = nodes_v.astype(jnp.int32)
    su, pu, bu = _prep(nu)
    sv, pv, bv = _prep(nv)
    tail_u = user_emb[TAIL_LO:]
    tail_v = item_emb[TAIL_LO:]
    stag_u = _sc_stream_gather(su, pu, bu, user_emb.T, tail_u)
    stag_v = _sc_stream_gather(sv, pv, bv, item_emb.T, tail_v)
    out = _sc_dot(stag_u, stag_v)
    return out.reshape(BATCH, 1, 1)
